# baseline (speedup 1.0000x reference)
import functools

import jax
import jax.numpy as jnp
from jax import lax
from jax.experimental import pallas as pl
from jax.experimental.pallas import tpu as pltpu
from jax.experimental.pallas import tpu_sc as plsc

_B = 4096
_D = 64
_L = 4
_K = 1024
_BB = 256
_NW = 32
_RWK = (_B * _L) // _NW
_CH = 16
_NCH = _RWK // _CH
_NEG_INF = float("-inf")


def _encode_block(x_ref, cbs2_ref, c2_ref, codes_ref):
    residual = x_ref[...].astype(jnp.bfloat16)
    c2 = c2_ref[...]
    lane = lax.broadcasted_iota(jnp.int32, (_BB, _K), 1)
    codes = []
    for l in range(_L):
        cb2 = cbs2_ref[l]
        r32 = residual.astype(jnp.float32)
        mm2 = lax.dot_general(
            residual, cb2, (((1,), (1,)), ((), ())),
            preferred_element_type=jnp.float32)
        r2 = jnp.sum(r32 * r32, axis=1, keepdims=True)
        d2 = (r2 - mm2) + c2[l][None, :]
        m = jnp.min(d2, axis=1, keepdims=True)
        cand = jnp.where(d2 == m, lane, _K)
        code = jnp.min(cand, axis=1, keepdims=True)
        codes.append(code)
        if l + 1 < _L:
            onehot = lane == code
            g = (0.5 * lax.dot_general(
                onehot.astype(jnp.bfloat16), cb2, (((1,), (0,)), ((), ())),
                preferred_element_type=jnp.float32)).astype(jnp.bfloat16)
            residual = residual - g
    codes_ref[...] = jnp.concatenate(codes, axis=1)


def _encode(x, cbs, c2):
    return pl.pallas_call(
        _encode_block,
        grid=(_B // _BB,),
        in_specs=[
            pl.BlockSpec((_BB, _D), lambda i: (i, 0)),
            pl.BlockSpec((_L, _K, _D), lambda i: (0, 0, 0)),
            pl.BlockSpec((_L, _K), lambda i: (0, 0)),
        ],
        out_specs=pl.BlockSpec((_BB, _L), lambda i: (i, 0)),
        out_shape=jax.ShapeDtypeStruct((_B, _L), jnp.int32),
    )(x, cbs, c2)


@functools.partial(
    pl.kernel,
    mesh=plsc.VectorSubcoreMesh(
        core_axis_name="c", subcore_axis_name="s", num_cores=2),
    out_type=[
        jax.ShapeDtypeStruct((_B * _L, _K), jnp.float32),
        jax.ShapeDtypeStruct((_B * _L, _K), jnp.float32),
    ],
    scratch_types=[
        pltpu.VMEM((_RWK,), jnp.int32),
        pltpu.VMEM((2, _CH, _K), jnp.float32),
        pltpu.VMEM((2, _CH, _K), jnp.float32),
        pltpu.SemaphoreType.DMA,
        pltpu.SemaphoreType.DMA,
        pltpu.SemaphoreType.DMA,
        pltpu.SemaphoreType.DMA,
    ],
)
def _sc_materialize(codes_hbm, ltab_hbm, ptab_hbm, logits_hbm, probs_hbm,
                    cbuf, lrows, prows, glsem, gpsem, slsem, spsem):
    wid = lax.axis_index("s") * 2 + lax.axis_index("c")
    base = wid * _RWK

    pltpu.sync_copy(codes_hbm.at[pl.ds(base, _RWK)], cbuf)

    def gathers(k):
        slot = k % 2
        idx = cbuf.at[pl.ds(k * _CH, _CH)]
        gl = pltpu.make_async_copy(ltab_hbm.at[idx], lrows.at[slot], glsem)
        gp = pltpu.make_async_copy(ptab_hbm.at[idx], prows.at[slot], gpsem)
        return gl, gp

    def stores(k):
        slot = k % 2
        rs = base + k * _CH
        sl = pltpu.make_async_copy(
            lrows.at[slot], logits_hbm.at[pl.ds(rs, _CH)], slsem)
        sp = pltpu.make_async_copy(
            prows.at[slot], probs_hbm.at[pl.ds(rs, _CH)], spsem)
        return sl, sp

    gl, gp = gathers(0)
    gl.start()
    gp.start()
    for k in range(_NCH):
        gl, gp = gathers(k)
        gl.wait()
        gp.wait()
        if k >= 1:
            sl, sp = stores(k - 1)
            sl.wait()
            sp.wait()
        if k + 1 < _NCH:
            ngl, ngp = gathers(k + 1)
            ngl.start()
            ngp.start()
        sl, sp = stores(k)
        sl.start()
        sp.start()
    sl, sp = stores(_NCH - 1)
    sl.wait()
    sp.wait()


def kernel(x, codebooks):
    cbs = codebooks.astype(jnp.bfloat16)
    c32 = cbs.astype(jnp.float32)
    c2 = jnp.sum(c32 * c32, axis=-1)
    codes = _encode(x, cbs * jnp.bfloat16(2.0), c2)
    eye = jnp.eye(_K, dtype=jnp.float32)
    ltab = jnp.where(eye > 0.0, eye, _NEG_INF)
    logits2, probs2 = _sc_materialize(codes.reshape(_B * _L), ltab, eye)
    return logits2.reshape(_B, _L, _K), probs2.reshape(_B, _L, _K)


# speedup vs baseline: 1.0033x; 1.0033x over previous
import functools

import jax
import jax.numpy as jnp
from jax import lax
from jax.experimental import pallas as pl
from jax.experimental.pallas import tpu as pltpu
from jax.experimental.pallas import tpu_sc as plsc

_B = 4096
_D = 64
_L = 4
_K = 1024
_BB = 256
_NW = 32
_RWK = (_B * _L) // _NW
_CH = 16
_NCH = _RWK // _CH
_NEG_INF = float("-inf")


def _encode_block(x_ref, cbs_ref, c2_ref, codes_ref):
    residual = x_ref[...].astype(jnp.bfloat16)
    c2 = c2_ref[...]
    lane = lax.broadcasted_iota(jnp.int32, (_BB, _K), 1)
    codes = []
    for l in range(_L):
        cb = cbs_ref[l]
        r32 = residual.astype(jnp.float32)
        mm = lax.dot_general(
            residual, cb, (((1,), (1,)), ((), ())),
            preferred_element_type=jnp.float32)
        r2 = jnp.sum(r32 * r32, axis=1, keepdims=True)
        d2 = (r2 - 2.0 * mm) + c2[l][None, :]
        m = jnp.min(d2, axis=1, keepdims=True)
        cand = jnp.where(d2 == m, lane, _K)
        code = jnp.min(cand, axis=1, keepdims=True)
        codes.append(code)
        if l + 1 < _L:
            onehot = lane == code
            g = lax.dot_general(
                onehot.astype(jnp.bfloat16), cb, (((1,), (0,)), ((), ())),
                preferred_element_type=jnp.float32).astype(jnp.bfloat16)
            residual = residual - g
    codes_ref[...] = jnp.concatenate(codes, axis=1)


def _encode(x, cbs, c2):
    return pl.pallas_call(
        _encode_block,
        grid=(_B // _BB,),
        in_specs=[
            pl.BlockSpec((_BB, _D), lambda i: (i, 0)),
            pl.BlockSpec((_L, _K, _D), lambda i: (0, 0, 0)),
            pl.BlockSpec((_L, _K), lambda i: (0, 0)),
        ],
        out_specs=pl.BlockSpec((_BB, _L), lambda i: (i, 0)),
        out_shape=jax.ShapeDtypeStruct((_B, _L), jnp.int32),
    )(x, cbs, c2)


@functools.partial(
    pl.kernel,
    mesh=plsc.VectorSubcoreMesh(core_axis_name="c", subcore_axis_name="s"),
    out_type=[
        jax.ShapeDtypeStruct((_B * _L, _K), jnp.float32),
        jax.ShapeDtypeStruct((_B * _L, _K), jnp.float32),
    ],
    scratch_types=[
        pltpu.VMEM((_RWK,), jnp.int32),
        pltpu.VMEM((2, _CH, _K), jnp.float32),
        pltpu.VMEM((2, _CH, _K), jnp.float32),
        pltpu.SemaphoreType.DMA,
        pltpu.SemaphoreType.DMA,
        pltpu.SemaphoreType.DMA,
        pltpu.SemaphoreType.DMA,
    ],
)
def _sc_materialize(codes_hbm, ltab_hbm, ptab_hbm, logits_hbm, probs_hbm,
                    cbuf, lrows, prows, glsem, gpsem, slsem, spsem):
    wid = lax.axis_index("s") * 2 + lax.axis_index("c")
    base = wid * _RWK

    pltpu.sync_copy(codes_hbm.at[pl.ds(base, _RWK)], cbuf)

    def gathers(k):
        slot = k % 2
        idx = cbuf.at[pl.ds(k * _CH, _CH)]
        gl = pltpu.make_async_copy(ltab_hbm.at[idx], lrows.at[slot], glsem)
        gp = pltpu.make_async_copy(ptab_hbm.at[idx], prows.at[slot], gpsem)
        return gl, gp

    def stores(k):
        slot = k % 2
        rs = base + k * _CH
        sl = pltpu.make_async_copy(
            lrows.at[slot], logits_hbm.at[pl.ds(rs, _CH)], slsem)
        sp = pltpu.make_async_copy(
            prows.at[slot], probs_hbm.at[pl.ds(rs, _CH)], spsem)
        return sl, sp

    gl, gp = gathers(0)
    gl.start()
    gp.start()
    for k in range(_NCH):
        gl, gp = gathers(k)
        gl.wait()
        gp.wait()
        if k >= 1:
            sl, sp = stores(k - 1)
            sl.wait()
            sp.wait()
        if k + 1 < _NCH:
            ngl, ngp = gathers(k + 1)
            ngl.start()
            ngp.start()
        sl, sp = stores(k)
        sl.start()
        sp.start()
    sl, sp = stores(_NCH - 1)
    sl.wait()
    sp.wait()


def kernel(x, codebooks):
    cbs = codebooks.astype(jnp.bfloat16)
    c32 = cbs.astype(jnp.float32)
    c2 = jnp.sum(c32 * c32, axis=-1)
    codes = _encode(x, cbs, c2)
    eye = jnp.eye(_K, dtype=jnp.float32)
    ltab = jnp.where(eye > 0.0, eye, _NEG_INF)
    logits2, probs2 = _sc_materialize(codes.reshape(_B * _L), ltab, eye)
    return logits2.reshape(_B, _L, _K), probs2.reshape(_B, _L, _K)
